# baseline (speedup 1.0000x reference)
import functools

import jax
import jax.numpy as jnp
from jax import lax
from jax.experimental import pallas as pl
from jax.experimental.pallas import tpu as pltpu
from jax.experimental.pallas import tpu_sc as plsc

_LANES = 16
_NBUF = 2
_CH = 128


def _sc_embed(idx_flat, tok_pairs, pos_pairs, t_period, d):
    n = idx_flat.shape[0]
    nw = 32
    per_w = n // nw
    ch = _CH
    n_rounds = (per_w // ch) // _NBUF
    n_grp = ch // _LANES
    d_sl = d // _LANES
    half_t = t_period // 2
    pos_rows = half_t + (-half_t) % 8

    mesh = plsc.VectorSubcoreMesh(core_axis_name="c", subcore_axis_name="s")

    @functools.partial(
        pl.kernel,
        out_type=jax.ShapeDtypeStruct((n // 2, 2 * d), jnp.float32),
        mesh=mesh,
        scratch_types=(
            [pltpu.VMEM((pos_rows, 2 * d), jnp.float32)]
            + [pltpu.VMEM((ch,), jnp.int32) for _ in range(_NBUF)]
            + [pltpu.VMEM((ch,), jnp.int32) for _ in range(_NBUF)]
            + [pltpu.VMEM((ch,), jnp.int32) for _ in range(_NBUF)]
            + [pltpu.VMEM((ch, 2 * d), jnp.float32) for _ in range(_NBUF)]
            + [pltpu.VMEM((ch // 2, 2 * d), jnp.float32) for _ in range(_NBUF)]
            + [pltpu.SemaphoreType.DMA for _ in range(3 * _NBUF)]
        ),
    )
    def k(idx_hbm, tok_hbm, pos_hbm, out_hbm, posv, *rest):
        idxv = rest[:_NBUF]
        pbuf = rest[_NBUF:2 * _NBUF]
        obase = rest[2 * _NBUF:3 * _NBUF]
        bufs = rest[3 * _NBUF:4 * _NBUF]
        obufs = rest[4 * _NBUF:5 * _NBUF]
        sem_i = rest[5 * _NBUF:6 * _NBUF]
        sem_g = rest[6 * _NBUF:7 * _NBUF]
        sem_s = rest[7 * _NBUF:]
        wid = lax.axis_index("s") * 2 + lax.axis_index("c")
        base = wid * per_w
        pltpu.sync_copy(pos_hbm.at[pl.ds(0, pos_rows)], posv)

        def round_body(g, carry):
            idx_loads, gathers = [], []
            for b in range(_NBUF):
                u = g * _NBUF + b

                @pl.when(g > 0)
                def _drain():
                    pltpu.make_async_copy(
                        obufs[b], out_hbm.at[pl.ds(0, ch // 2)],
                        sem_s[b]).wait()

                idx_loads.append(pltpu.async_copy(
                    idx_hbm.at[pl.ds(pl.multiple_of(base + u * ch, 8), ch)],
                    idxv[b], sem_i[b]))
            for b in range(_NBUF):
                idx_loads[b].wait()

                def prep(i, c2):
                    v = idxv[b][pl.ds(i * _LANES, _LANES)]
                    pbuf[b][pl.ds(i * _LANES, _LANES)] = (
                        lax.shift_right_logical(v, 1))
                    obase[b][pl.ds(i * _LANES, _LANES)] = lax.shift_left(
                        lax.bitwise_and(v, 1), 6)
                    return c2

                lax.fori_loop(0, n_grp, prep, 0)
                gathers.append(pltpu.async_copy(
                    tok_hbm.at[pbuf[b]], bufs[b], sem_g[b]))
            for b in range(_NBUF):
                u = g * _NBUF + b
                gathers[b].wait()
                pos0 = (u * (ch // 2)) % half_t

                def sel(i, c2):
                    ov = obase[b][pl.ds(i * _LANES, _LANES)]
                    for r in range(_LANES):
                        j = i * _LANES + r
                        jj = i * (_LANES // 2) + r // 2
                        col = (r % 2) * d
                        o = ov[r]
                        jp = lax.rem(pos0 + jj, half_t)
                        for s in range(d_sl):
                            obufs[b][jj, pl.ds(col + s * _LANES, _LANES)] = (
                                bufs[b][j, pl.ds(o + s * _LANES, _LANES)]
                                + posv[jp, pl.ds(col + s * _LANES, _LANES)])
                    return c2

                lax.fori_loop(0, n_grp, sel, 0)
                pltpu.async_copy(
                    obufs[b],
                    out_hbm.at[pl.ds(
                        pl.multiple_of((base + u * ch) // 2, 8), ch // 2)],
                    sem_s[b])
            return carry

        lax.fori_loop(0, n_rounds, round_body, 0)
        for b in range(_NBUF):
            pltpu.make_async_copy(
                obufs[b], out_hbm.at[pl.ds(0, ch // 2)], sem_s[b]).wait()

    return k(idx_flat, tok_pairs, pos_pairs)


def kernel(idx, tok_emb, pos_emb):
    b, t = idx.shape
    v, d = tok_emb.shape
    flat = idx.reshape(b * t).astype(jnp.int32)
    tok_pairs = tok_emb.reshape(v // 2, 2 * d)
    pos_pairs = pos_emb.reshape(pos_emb.shape[0] // 2, 2 * d)
    out = _sc_embed(flat, tok_pairs, pos_pairs, t, d)
    return out.reshape(b, t, d)


# speedup vs baseline: 1.7469x; 1.7469x over previous
import functools

import jax
import jax.numpy as jnp
from jax import lax
from jax.experimental import pallas as pl
from jax.experimental.pallas import tpu as pltpu
from jax.experimental.pallas import tpu_sc as plsc

_LANES = 16
_NBUF = 4


def _sc_embed(idx_flat, tok_emb, pos_emb, t_period):
    n = idx_flat.shape[0]
    d = tok_emb.shape[1]
    nw = 32
    per_w = n // nw
    ch = t_period
    n_chunks = per_w // ch
    n_rounds = n_chunks // _NBUF
    n_grp = ch // _LANES
    d_sl = d // _LANES

    mesh = plsc.VectorSubcoreMesh(core_axis_name="c", subcore_axis_name="s")

    @functools.partial(
        pl.kernel,
        out_type=jax.ShapeDtypeStruct((n, d), jnp.float32),
        mesh=mesh,
        compiler_params=pltpu.CompilerParams(use_tc_tiling_on_sc=False),
        scratch_types=(
            [pltpu.VMEM((per_w,), jnp.int32),
             pltpu.VMEM((ch, d), jnp.float32)]
            + [pltpu.VMEM((ch, d), jnp.float32) for _ in range(_NBUF)]
            + [pltpu.SemaphoreType.DMA for _ in range(2 * _NBUF)]
        ),
    )
    def k(idx_hbm, tok_hbm, pos_hbm, out_hbm, idx_all, posv, *rest):
        bufs = rest[:_NBUF]
        sem_g = rest[_NBUF:2 * _NBUF]
        sem_s = rest[2 * _NBUF:]
        wid = lax.axis_index("s") * 2 + lax.axis_index("c")
        base = wid * per_w
        pltpu.sync_copy(pos_hbm.at[pl.ds(0, ch)], posv)
        pltpu.sync_copy(idx_hbm.at[pl.ds(base, per_w)], idx_all)

        def round_body(g, carry):
            gathers = []
            for b in range(_NBUF):
                u = g * _NBUF + b

                @pl.when(g > 0)
                def _drain():
                    pltpu.make_async_copy(
                        bufs[b], out_hbm.at[pl.ds(0, ch)], sem_s[b]).wait()

                gathers.append(pltpu.async_copy(
                    tok_hbm.at[idx_all.at[pl.ds(u * ch, ch)]],
                    bufs[b], sem_g[b]))
            for b in range(_NBUF):
                u = g * _NBUF + b
                gathers[b].wait()

                def add_rows(i, c2):
                    for r in range(_LANES):
                        j = i * _LANES + r
                        for s in range(d_sl):
                            sl = pl.ds(s * _LANES, _LANES)
                            bufs[b][j, sl] = bufs[b][j, sl] + posv[j, sl]
                    return c2

                lax.fori_loop(0, n_grp, add_rows, 0)
                for j in range(n_grp * _LANES, ch):
                    for s in range(d_sl):
                        sl = pl.ds(s * _LANES, _LANES)
                        bufs[b][j, sl] = bufs[b][j, sl] + posv[j, sl]
                pltpu.async_copy(
                    bufs[b], out_hbm.at[pl.ds(base + u * ch, ch)], sem_s[b])
            return carry

        lax.fori_loop(0, n_rounds, round_body, 0)
        for b in range(_NBUF):
            pltpu.make_async_copy(
                bufs[b], out_hbm.at[pl.ds(0, ch)], sem_s[b]).wait()

    return k(idx_flat, tok_emb, pos_emb)


def kernel(idx, tok_emb, pos_emb):
    b, t = idx.shape
    d = tok_emb.shape[1]
    flat = idx.reshape(b * t).astype(jnp.int32)
    out = _sc_embed(flat, tok_emb, pos_emb, t)
    return out.reshape(b, t, d)
